# Initial kernel scaffold; baseline (speedup 1.0000x reference)
import jax
import jax.numpy as jnp
from jax.experimental import pallas as pl


def kernel(fea, edge_index, edge_weight, W0, Ws0, b0, g0, be0, W1, Ws1, b1, g1, be1, W2, Ws2, b2, g2, be2, W3, Ws3, b3, g3, be3):
    raise NotImplementedError("write your pallas kernel here")


# speedup vs baseline: 3.6001x; 3.6001x over previous
import functools

import jax
import jax.numpy as jnp
from jax import lax
from jax.experimental import pallas as pl
from jax.experimental.pallas import tpu as pltpu
from jax.experimental.pallas import tpu_sc as plsc

N = 10000
E = 320000
NC = 2
NS = 16
L = 16
NW = NC * NS
EPW = E // NW
T = 80
NCHUNK = EPW // T
BR = 200
NB = N // BR


@functools.partial(jax.jit, static_argnames=("d",))
def _sc_aggregate(sup, src, dst, ew, *, d):
  mesh = plsc.VectorSubcoreMesh(core_axis_name="c", subcore_axis_name="s")

  @functools.partial(
      pl.kernel,
      mesh=mesh,
      out_type=jax.ShapeDtypeStruct((NC * N, d), jnp.float32),
      scratch_types=[
          pltpu.VMEM((T,), jnp.int32),
          pltpu.VMEM((T,), jnp.int32),
          pltpu.VMEM((T,), jnp.float32),
          pltpu.VMEM((T, d), jnp.float32),
          pltpu.VMEM((BR, d), jnp.float32),
          pltpu.VMEM_SHARED((N, d), jnp.float32),
          pltpu.SemaphoreType.DMA,
      ],
  )
  def agg(sup_hbm, src_hbm, dst_hbm, ew_hbm, out_hbm,
          src_v, dst_v, ew_v, rows_v, zb_v, acc_sh, sem):
    cid = lax.axis_index("c")
    sid = lax.axis_index("s")
    wid = sid * NC + cid

    zero = jnp.zeros((L,), jnp.float32)

    @pl.loop(0, BR)
    def _zb(r):
      for j in range(d // L):
        zb_v[r, pl.ds(j * L, L)] = zero

    for b in range(NB):
      @pl.when(b % NS == sid)
      def _z():
        pltpu.sync_copy(zb_v, acc_sh.at[pl.ds(b * BR, BR)])
    plsc.subcore_barrier()

    ebase = wid * EPW

    @pl.loop(0, NCHUNK)
    def _chunk(k):
      off = ebase + k * T
      pltpu.sync_copy(src_hbm.at[pl.ds(off, T)], src_v)
      pltpu.sync_copy(dst_hbm.at[pl.ds(off, T)], dst_v)
      pltpu.sync_copy(ew_hbm.at[pl.ds(off, T)], ew_v)
      pltpu.async_copy(sup_hbm.at[src_v], rows_v, sem).wait()

      @pl.loop(0, T // L)
      def _grp(gi):
        ew_vec = ew_v[pl.ds(gi * L, L)]
        for i in range(L):
          w = jnp.full((L,), ew_vec[i])
          t = gi * L + i
          for j in range(d // L):
            sl = pl.ds(j * L, L)
            rows_v[t, sl] = rows_v[t, sl] * w

      pltpu.sync_copy(rows_v, acc_sh.at[dst_v], add=True)

    plsc.subcore_barrier()
    for b in range(NB):
      @pl.when(b % NS == sid)
      def _o():
        pltpu.sync_copy(acc_sh.at[pl.ds(b * BR, BR)],
                        out_hbm.at[pl.ds(cid * N + b * BR, BR)])

  return agg(sup, src, dst, ew).reshape(NC, N, d)


def _tc_matmul2(x, W, Ws, b):
  m, k = x.shape
  ds_ = W.shape[1]
  dl = Ws.shape[1]
  bm = 1000

  def body(x_ref, w_ref, ws_ref, b_ref, sup_ref, slf_ref):
    xb = x_ref[...]
    sup_ref[...] = jnp.dot(xb, w_ref[...], preferred_element_type=jnp.float32)
    slf_ref[...] = (jnp.dot(xb, ws_ref[...], preferred_element_type=jnp.float32)
                    + b_ref[...])

  return pl.pallas_call(
      body,
      grid=(m // bm,),
      in_specs=[
          pl.BlockSpec((bm, k), lambda i: (i, 0)),
          pl.BlockSpec((k, ds_), lambda i: (0, 0)),
          pl.BlockSpec((k, dl), lambda i: (0, 0)),
          pl.BlockSpec((1, dl), lambda i: (0, 0)),
      ],
      out_specs=[
          pl.BlockSpec((bm, ds_), lambda i: (i, 0)),
          pl.BlockSpec((bm, dl), lambda i: (i, 0)),
      ],
      out_shape=[jax.ShapeDtypeStruct((m, ds_), jnp.float32),
                 jax.ShapeDtypeStruct((m, dl), jnp.float32)],
  )(x, W, Ws, b.reshape(1, dl))


def _tc_combine(parts, slf, g, be, resid=None, softmax=False):
  n, d = slf.shape

  def body(*refs):
    if resid is None:
      p_ref, slf_ref, g_ref, be_ref, out_ref = refs
      r_ref = None
    else:
      p_ref, slf_ref, g_ref, be_ref, r_ref, out_ref = refs
    o = p_ref[0][:, :d] + p_ref[1][:, :d] + slf_ref[...]
    mean = jnp.mean(o, axis=0, keepdims=True)
    var = jnp.mean(jnp.square(o - mean), axis=0, keepdims=True)
    o = (o - mean) * lax.rsqrt(var + 1e-5) * g_ref[...] + be_ref[...]
    if r_ref is not None:
      o = o + r_ref[...]
    if softmax:
      mx = jnp.max(o, axis=1, keepdims=True)
      o = o - mx
      o = o - jnp.log(jnp.sum(jnp.exp(o), axis=1, keepdims=True))
    out_ref[...] = o

  args = [parts, slf, g.reshape(1, d), be.reshape(1, d)]
  if resid is not None:
    args.append(resid)
  return pl.pallas_call(
      body,
      out_shape=jax.ShapeDtypeStruct((n, d), jnp.float32),
  )(*args)


def kernel(fea, edge_index, edge_weight,
           W0, Ws0, b0, g0, be0,
           W1, Ws1, b1, g1, be1,
           W2, Ws2, b2, g2, be2,
           W3, Ws3, b3, g3, be3):
  src = edge_index[0]
  dst = edge_index[1]
  ew = edge_weight

  sup, slf = _tc_matmul2(fea, W0, Ws0, b0)
  parts = _sc_aggregate(sup, src, dst, ew, d=128)
  x0 = _tc_combine(parts, slf, g0, be0)

  sup, slf = _tc_matmul2(x0, W1, Ws1, b1)
  parts = _sc_aggregate(sup, src, dst, ew, d=128)
  h = _tc_combine(parts, slf, g1, be1)

  sup, slf = _tc_matmul2(h, W2, Ws2, b2)
  parts = _sc_aggregate(sup, src, dst, ew, d=128)
  x = _tc_combine(parts, slf, g2, be2, resid=x0)

  W3p = jnp.pad(W3, ((0, 0), (0, 128 - W3.shape[1])))
  sup, slf = _tc_matmul2(x, W3p, Ws3, b3)
  parts = _sc_aggregate(sup, src, dst, ew, d=128)
  return _tc_combine(parts, slf, g3, be3, softmax=True)


# speedup vs baseline: 7.5284x; 2.0912x over previous
import functools

import jax
import jax.numpy as jnp
from jax import lax
from jax.experimental import pallas as pl
from jax.experimental.pallas import tpu as pltpu
from jax.experimental.pallas import tpu_sc as plsc

N = 10000
E = 320000
NC = 2
NS = 16
L = 16
NW = NC * NS
EPW = E // NW
T = 80
NCHUNK = EPW // T
BR = 200
NB = N // BR


@functools.partial(jax.jit, static_argnames=("d",))
def _sc_aggregate(sup, src, dst, ew, *, d):
  mesh = plsc.VectorSubcoreMesh(core_axis_name="c", subcore_axis_name="s")

  @functools.partial(
      pl.kernel,
      mesh=mesh,
      out_type=jax.ShapeDtypeStruct((NC * N, d), jnp.float32),
      scratch_types=[
          pltpu.VMEM((T,), jnp.int32),
          pltpu.VMEM((T,), jnp.int32),
          pltpu.VMEM((T,), jnp.int32),
          pltpu.VMEM((T,), jnp.int32),
          pltpu.VMEM((T,), jnp.float32),
          pltpu.VMEM((T,), jnp.float32),
          pltpu.VMEM((T, d), jnp.float32),
          pltpu.VMEM((T, d), jnp.float32),
          pltpu.VMEM_SHARED((N, d), jnp.float32),
          pltpu.SemaphoreType.DMA,
          pltpu.SemaphoreType.DMA,
          pltpu.SemaphoreType.DMA,
          pltpu.SemaphoreType.DMA,
      ],
  )
  def agg(sup_hbm, src_hbm, dst_hbm, ew_hbm, out_hbm,
          src0, src1, dst0, dst1, ew0, ew1, rows0, rows1, acc_sh,
          isem0, isem1, gsem0, gsem1):
    cid = lax.axis_index("c")
    sid = lax.axis_index("s")
    wid = sid * NC + cid
    ebase = wid * EPW
    srcs = (src0, src1)
    dsts = (dst0, dst1)
    ews = (ew0, ew1)
    rows = (rows0, rows1)
    isems = (isem0, isem1)
    gsems = (gsem0, gsem1)

    def idx_load(k, b, wait):
      off = ebase + k * T
      for hbm, buf in ((src_hbm, srcs[b]), (dst_hbm, dsts[b]),
                       (ew_hbm, ews[b])):
        c = pltpu.async_copy(hbm.at[pl.ds(off, T)], buf, isems[b])
        if wait:
          c.wait()

    def idx_wait(k, b):
      off = ebase + k * T
      for hbm, buf in ((src_hbm, srcs[b]), (dst_hbm, dsts[b]),
                       (ew_hbm, ews[b])):
        pltpu.make_async_copy(hbm.at[pl.ds(off, T)], buf, isems[b]).wait()

    zero = jnp.zeros((L,), jnp.float32)

    @pl.loop(0, T)
    def _zb(r):
      for j in range(d // L):
        rows1[r, pl.ds(j * L, L)] = zero

    for b in range(N // T):
      @pl.when(b % NS == sid)
      def _z():
        pltpu.sync_copy(rows1, acc_sh.at[pl.ds(b * T, T)])

    idx_load(0, 0, True)
    pltpu.async_copy(sup_hbm.at[src0], rows0, gsem0)
    idx_load(1, 1, False)
    plsc.subcore_barrier()

    def scale(rows_ref, ew_ref):
      @pl.loop(0, T // L)
      def _grp(gi):
        ew_vec = ew_ref[pl.ds(gi * L, L)]
        for i in range(L):
          t = gi * L + i
          w = jnp.full((L,), ew_vec[i])
          for j in range(d // L):
            sl = pl.ds(j * L, L)
            rows_ref[t, sl] = rows_ref[t, sl] * w

    def stage(k, b):
      nb = 1 - b

      @pl.when(k + 1 < NCHUNK)
      def _pf():
        idx_wait(k + 1, nb)
        pltpu.async_copy(sup_hbm.at[srcs[nb]], rows[nb], gsems[nb])
      pltpu.make_async_copy(sup_hbm.at[srcs[b]], rows[b], gsems[b]).wait()
      scale(rows[b], ews[b])
      pltpu.sync_copy(rows[b], acc_sh.at[dsts[b]], add=True)

      @pl.when(k + 2 < NCHUNK)
      def _pi():
        idx_load(k + 2, b, False)

    @pl.loop(0, NCHUNK, step=2)
    def _chunk(k):
      stage(k, 0)

      @pl.when(k + 1 < NCHUNK)
      def _b1():
        stage(k + 1, 1)

    plsc.subcore_barrier()
    for b in range(NB):
      @pl.when(b % NS == sid)
      def _o():
        pltpu.sync_copy(acc_sh.at[pl.ds(b * BR, BR)],
                        out_hbm.at[pl.ds(cid * N + b * BR, BR)])

  return agg(sup, src, dst, ew).reshape(NC, N, d)


def _tc_matmul2(x, W, Ws, b):
  m, k = x.shape
  ds_ = W.shape[1]
  dl = Ws.shape[1]
  bm = 1000

  def body(x_ref, w_ref, ws_ref, b_ref, sup_ref, slf_ref):
    xb = x_ref[...]
    sup_ref[...] = jnp.dot(xb, w_ref[...], preferred_element_type=jnp.float32)
    slf_ref[...] = (jnp.dot(xb, ws_ref[...], preferred_element_type=jnp.float32)
                    + b_ref[...])

  return pl.pallas_call(
      body,
      grid=(m // bm,),
      in_specs=[
          pl.BlockSpec((bm, k), lambda i: (i, 0)),
          pl.BlockSpec((k, ds_), lambda i: (0, 0)),
          pl.BlockSpec((k, dl), lambda i: (0, 0)),
          pl.BlockSpec((1, dl), lambda i: (0, 0)),
      ],
      out_specs=[
          pl.BlockSpec((bm, ds_), lambda i: (i, 0)),
          pl.BlockSpec((bm, dl), lambda i: (i, 0)),
      ],
      out_shape=[jax.ShapeDtypeStruct((m, ds_), jnp.float32),
                 jax.ShapeDtypeStruct((m, dl), jnp.float32)],
  )(x, W, Ws, b.reshape(1, dl))


def _tc_combine(parts, slf, g, be, resid=None, softmax=False):
  n, d = slf.shape

  def body(*refs):
    if resid is None:
      p_ref, slf_ref, g_ref, be_ref, out_ref = refs
      r_ref = None
    else:
      p_ref, slf_ref, g_ref, be_ref, r_ref, out_ref = refs
    o = p_ref[0][:, :d] + p_ref[1][:, :d] + slf_ref[...]
    mean = jnp.mean(o, axis=0, keepdims=True)
    var = jnp.mean(jnp.square(o - mean), axis=0, keepdims=True)
    o = (o - mean) * lax.rsqrt(var + 1e-5) * g_ref[...] + be_ref[...]
    if r_ref is not None:
      o = o + r_ref[...]
    if softmax:
      mx = jnp.max(o, axis=1, keepdims=True)
      o = o - mx
      o = o - jnp.log(jnp.sum(jnp.exp(o), axis=1, keepdims=True))
    out_ref[...] = o

  args = [parts, slf, g.reshape(1, d), be.reshape(1, d)]
  if resid is not None:
    args.append(resid)
  return pl.pallas_call(
      body,
      out_shape=jax.ShapeDtypeStruct((n, d), jnp.float32),
  )(*args)


def kernel(fea, edge_index, edge_weight,
           W0, Ws0, b0, g0, be0,
           W1, Ws1, b1, g1, be1,
           W2, Ws2, b2, g2, be2,
           W3, Ws3, b3, g3, be3):
  src = edge_index[0]
  dst = edge_index[1]
  ew = edge_weight

  sup, slf = _tc_matmul2(fea, W0, Ws0, b0)
  parts = _sc_aggregate(sup, src, dst, ew, d=128)
  x0 = _tc_combine(parts, slf, g0, be0)

  sup, slf = _tc_matmul2(x0, W1, Ws1, b1)
  parts = _sc_aggregate(sup, src, dst, ew, d=128)
  h = _tc_combine(parts, slf, g1, be1)

  sup, slf = _tc_matmul2(h, W2, Ws2, b2)
  parts = _sc_aggregate(sup, src, dst, ew, d=128)
  x = _tc_combine(parts, slf, g2, be2, resid=x0)

  W3p = jnp.pad(W3, ((0, 0), (0, 128 - W3.shape[1])))
  sup, slf = _tc_matmul2(x, W3p, Ws3, b3)
  parts = _sc_aggregate(sup, src, dst, ew, d=128)
  return _tc_combine(parts, slf, g3, be3, softmax=True)


# speedup vs baseline: 8.9957x; 1.1949x over previous
import functools

import jax
import jax.numpy as jnp
from jax import lax
from jax.experimental import pallas as pl
from jax.experimental.pallas import tpu as pltpu
from jax.experimental.pallas import tpu_sc as plsc

N = 10000
E = 320000
NC = 2
NS = 16
L = 16
NW = NC * NS
EPW = E // NW
T = 80
NCHUNK = EPW // T
BR = 200
NB = N // BR


@functools.partial(jax.jit, static_argnames=("d",))
def _sc_aggregate(sup, src, dst, ew, *, d):
  mesh = plsc.VectorSubcoreMesh(core_axis_name="c", subcore_axis_name="s")

  @functools.partial(
      pl.kernel,
      mesh=mesh,
      out_type=jax.ShapeDtypeStruct((NC * N, d), jnp.float32),
      scratch_types=[
          pltpu.VMEM((T,), jnp.int32),
          pltpu.VMEM((T,), jnp.int32),
          pltpu.VMEM((T,), jnp.int32),
          pltpu.VMEM((T,), jnp.int32),
          pltpu.VMEM((T,), jnp.float32),
          pltpu.VMEM((T,), jnp.float32),
          pltpu.VMEM((T,), jnp.int32),
          pltpu.VMEM((T,), jnp.int32),
          pltpu.VMEM((T, d), jnp.float32),
          pltpu.VMEM((T, d), jnp.float32),
          pltpu.VMEM_SHARED((N, d), jnp.float32),
          pltpu.SemaphoreType.DMA,
          pltpu.SemaphoreType.DMA,
          pltpu.SemaphoreType.DMA,
          pltpu.SemaphoreType.DMA,
          pltpu.SemaphoreType.DMA,
          pltpu.SemaphoreType.DMA,
      ],
  )
  def agg(sup_hbm, src_hbm, dst_hbm, ew_hbm, out_hbm,
          src0, src1, dst0, dst1, ew0, ew1, sdst0, sdst1,
          rows0, rows1, acc_sh,
          isem0, isem1, gsem0, gsem1, ssem0, ssem1):
    cid = lax.axis_index("c")
    sid = lax.axis_index("s")
    wid = sid * NC + cid
    ebase = wid * EPW
    srcs = (src0, src1)
    dsts = (dst0, dst1)
    ews = (ew0, ew1)
    sdsts = (sdst0, sdst1)
    rows = (rows0, rows1)
    isems = (isem0, isem1)
    gsems = (gsem0, gsem1)
    ssems = (ssem0, ssem1)

    def idx_load(k, b, wait):
      off = ebase + k * T
      for hbm, buf in ((src_hbm, srcs[b]), (dst_hbm, dsts[b]),
                       (ew_hbm, ews[b])):
        c = pltpu.async_copy(hbm.at[pl.ds(off, T)], buf, isems[b])
        if wait:
          c.wait()

    def idx_wait(k, b):
      off = ebase + k * T
      for hbm, buf in ((src_hbm, srcs[b]), (dst_hbm, dsts[b]),
                       (ew_hbm, ews[b])):
        pltpu.make_async_copy(hbm.at[pl.ds(off, T)], buf, isems[b]).wait()

    zero = jnp.zeros((L,), jnp.float32)

    @pl.loop(0, T)
    def _zb(r):
      for j in range(d // L):
        rows1[r, pl.ds(j * L, L)] = zero

    for b in range(N // T):
      @pl.when(b % NS == sid)
      def _z():
        pltpu.sync_copy(rows1, acc_sh.at[pl.ds(b * T, T)])

    idx_load(0, 0, True)
    pltpu.async_copy(sup_hbm.at[src0], rows0, gsem0)
    idx_load(1, 1, False)
    plsc.subcore_barrier()

    def scale(rows_ref, ew_ref):
      @pl.loop(0, T // L)
      def _grp(gi):
        ew_vec = ew_ref[pl.ds(gi * L, L)]
        for i in range(L):
          t = gi * L + i
          w = jnp.full((L,), ew_vec[i])
          for j in range(d // L):
            sl = pl.ds(j * L, L)
            rows_ref[t, sl] = rows_ref[t, sl] * w

    def stage(k, b):
      nb = 1 - b

      @pl.when(k + 1 < NCHUNK)
      def _pf():
        idx_wait(k + 1, nb)
        @pl.when(k > 0)
        def _ws():
          pltpu.make_async_copy(rows[nb], acc_sh.at[sdsts[nb]],
                                ssems[nb]).wait()
        pltpu.async_copy(sup_hbm.at[srcs[nb]], rows[nb], gsems[nb])
      pltpu.make_async_copy(sup_hbm.at[srcs[b]], rows[b], gsems[b]).wait()
      scale(rows[b], ews[b])
      for gi in range(T // L):
        sl = pl.ds(gi * L, L)
        sdsts[b][sl] = dsts[b][sl]
      pltpu.async_copy(rows[b], acc_sh.at[sdsts[b]], ssems[b], add=True)

      @pl.when(k + 2 < NCHUNK)
      def _pi():
        idx_load(k + 2, b, False)

    @pl.loop(0, NCHUNK, step=2)
    def _chunk(k):
      stage(k, 0)

      @pl.when(k + 1 < NCHUNK)
      def _b1():
        stage(k + 1, 1)

    pltpu.make_async_copy(rows1, acc_sh.at[sdst1], ssem1).wait()
    pltpu.make_async_copy(rows0, acc_sh.at[sdst0], ssem0).wait()
    plsc.subcore_barrier()
    for b in range(NB):
      @pl.when(b % NS == sid)
      def _o():
        pltpu.sync_copy(acc_sh.at[pl.ds(b * BR, BR)],
                        out_hbm.at[pl.ds(cid * N + b * BR, BR)])

  return agg(sup, src, dst, ew).reshape(NC, N, d)


def _tc_matmul2(x, W, Ws, b):
  m, k = x.shape
  ds_ = W.shape[1]
  dl = Ws.shape[1]
  bm = 1000

  def body(x_ref, w_ref, ws_ref, b_ref, sup_ref, slf_ref):
    xb = x_ref[...]
    sup_ref[...] = jnp.dot(xb, w_ref[...], preferred_element_type=jnp.float32)
    slf_ref[...] = (jnp.dot(xb, ws_ref[...], preferred_element_type=jnp.float32)
                    + b_ref[...])

  return pl.pallas_call(
      body,
      grid=(m // bm,),
      in_specs=[
          pl.BlockSpec((bm, k), lambda i: (i, 0)),
          pl.BlockSpec((k, ds_), lambda i: (0, 0)),
          pl.BlockSpec((k, dl), lambda i: (0, 0)),
          pl.BlockSpec((1, dl), lambda i: (0, 0)),
      ],
      out_specs=[
          pl.BlockSpec((bm, ds_), lambda i: (i, 0)),
          pl.BlockSpec((bm, dl), lambda i: (i, 0)),
      ],
      out_shape=[jax.ShapeDtypeStruct((m, ds_), jnp.float32),
                 jax.ShapeDtypeStruct((m, dl), jnp.float32)],
  )(x, W, Ws, b.reshape(1, dl))


def _tc_combine(parts, slf, g, be, resid=None, softmax=False):
  n, d = slf.shape

  def body(*refs):
    if resid is None:
      p_ref, slf_ref, g_ref, be_ref, out_ref = refs
      r_ref = None
    else:
      p_ref, slf_ref, g_ref, be_ref, r_ref, out_ref = refs
    o = p_ref[0][:, :d] + p_ref[1][:, :d] + slf_ref[...]
    mean = jnp.mean(o, axis=0, keepdims=True)
    var = jnp.mean(jnp.square(o - mean), axis=0, keepdims=True)
    o = (o - mean) * lax.rsqrt(var + 1e-5) * g_ref[...] + be_ref[...]
    if r_ref is not None:
      o = o + r_ref[...]
    if softmax:
      mx = jnp.max(o, axis=1, keepdims=True)
      o = o - mx
      o = o - jnp.log(jnp.sum(jnp.exp(o), axis=1, keepdims=True))
    out_ref[...] = o

  args = [parts, slf, g.reshape(1, d), be.reshape(1, d)]
  if resid is not None:
    args.append(resid)
  return pl.pallas_call(
      body,
      out_shape=jax.ShapeDtypeStruct((n, d), jnp.float32),
  )(*args)


def kernel(fea, edge_index, edge_weight,
           W0, Ws0, b0, g0, be0,
           W1, Ws1, b1, g1, be1,
           W2, Ws2, b2, g2, be2,
           W3, Ws3, b3, g3, be3):
  src = edge_index[0]
  dst = edge_index[1]
  ew = edge_weight

  sup, slf = _tc_matmul2(fea, W0, Ws0, b0)
  parts = _sc_aggregate(sup, src, dst, ew, d=128)
  x0 = _tc_combine(parts, slf, g0, be0)

  sup, slf = _tc_matmul2(x0, W1, Ws1, b1)
  parts = _sc_aggregate(sup, src, dst, ew, d=128)
  h = _tc_combine(parts, slf, g1, be1)

  sup, slf = _tc_matmul2(h, W2, Ws2, b2)
  parts = _sc_aggregate(sup, src, dst, ew, d=128)
  x = _tc_combine(parts, slf, g2, be2, resid=x0)

  W3p = jnp.pad(W3, ((0, 0), (0, 128 - W3.shape[1])))
  sup, slf = _tc_matmul2(x, W3p, Ws3, b3)
  parts = _sc_aggregate(sup, src, dst, ew, d=128)
  return _tc_combine(parts, slf, g3, be3, softmax=True)


# speedup vs baseline: 9.2925x; 1.0330x over previous
import functools

import jax
import jax.numpy as jnp
from jax import lax
from jax.experimental import pallas as pl
from jax.experimental.pallas import tpu as pltpu
from jax.experimental.pallas import tpu_sc as plsc

N = 10000
E = 320000
NC = 2
NS = 16
L = 16
NW = NC * NS
EPW = E // NW
T = 80
NCHUNK = EPW // T
BR = 200
NB = N // BR


@functools.partial(jax.jit, static_argnames=("d",))
def _sc_aggregate(sup, src, dst, ew, *, d):
  mesh = plsc.VectorSubcoreMesh(core_axis_name="c", subcore_axis_name="s")

  @functools.partial(
      pl.kernel,
      mesh=mesh,
      out_type=jax.ShapeDtypeStruct((NC * N, d), jnp.float32),
      scratch_types=[
          pltpu.VMEM((T,), jnp.int32),
          pltpu.VMEM((T,), jnp.int32),
          pltpu.VMEM((T,), jnp.int32),
          pltpu.VMEM((T,), jnp.int32),
          pltpu.VMEM((T,), jnp.float32),
          pltpu.VMEM((T,), jnp.float32),
          pltpu.VMEM((T,), jnp.int32),
          pltpu.VMEM((T,), jnp.int32),
          pltpu.VMEM((T, d), jnp.float32),
          pltpu.VMEM((T, d), jnp.float32),
          pltpu.VMEM_SHARED((N, d), jnp.float32),
          pltpu.SemaphoreType.DMA,
          pltpu.SemaphoreType.DMA,
          pltpu.SemaphoreType.DMA,
          pltpu.SemaphoreType.DMA,
          pltpu.SemaphoreType.DMA,
          pltpu.SemaphoreType.DMA,
      ],
  )
  def agg(sup_hbm, src_hbm, dst_hbm, ew_hbm, out_hbm,
          src0, src1, dst0, dst1, ew0, ew1, sdst0, sdst1,
          rows0, rows1, acc_sh,
          isem0, isem1, gsem0, gsem1, ssem0, ssem1):
    cid = lax.axis_index("c")
    sid = lax.axis_index("s")
    wid = sid * NC + cid
    ebase = wid * EPW
    srcs = (src0, src1)
    dsts = (dst0, dst1)
    ews = (ew0, ew1)
    sdsts = (sdst0, sdst1)
    rows = (rows0, rows1)
    isems = (isem0, isem1)
    gsems = (gsem0, gsem1)
    ssems = (ssem0, ssem1)

    def idx_load(k, b, wait):
      off = ebase + k * T
      for hbm, buf in ((src_hbm, srcs[b]), (dst_hbm, dsts[b]),
                       (ew_hbm, ews[b])):
        c = pltpu.async_copy(hbm.at[pl.ds(off, T)], buf, isems[b])
        if wait:
          c.wait()

    def idx_wait(k, b):
      off = ebase + k * T
      for hbm, buf in ((src_hbm, srcs[b]), (dst_hbm, dsts[b]),
                       (ew_hbm, ews[b])):
        pltpu.make_async_copy(hbm.at[pl.ds(off, T)], buf, isems[b]).wait()

    zero = jnp.zeros((L,), jnp.float32)

    @pl.loop(0, T)
    def _zb(r):
      for j in range(d // L):
        rows1[r, pl.ds(j * L, L)] = zero

    for b in range(N // T):
      @pl.when(b % NS == sid)
      def _z():
        pltpu.sync_copy(rows1, acc_sh.at[pl.ds(b * T, T)])

    idx_load(0, 0, True)
    pltpu.async_copy(sup_hbm.at[src0], rows0, gsem0)
    idx_load(1, 1, False)
    plsc.subcore_barrier()

    def scale(rows_ref, ew_ref):
      @pl.loop(0, T // L)
      def _grp(gi):
        ew_vec = ew_ref[pl.ds(gi * L, L)]
        for i in range(L):
          t = gi * L + i
          w = jnp.full((L,), ew_vec[i])
          for j in range(d // L):
            sl = pl.ds(j * L, L)
            rows_ref[t, sl] = rows_ref[t, sl] * w

    def stage(k, b):
      nb = 1 - b

      @pl.when(k + 1 < NCHUNK)
      def _pf():
        idx_wait(k + 1, nb)
        @pl.when(k > 0)
        def _ws():
          pltpu.make_async_copy(rows[nb], acc_sh.at[sdsts[nb]],
                                ssems[nb]).wait()
        pltpu.async_copy(sup_hbm.at[srcs[nb]], rows[nb], gsems[nb])
      pltpu.make_async_copy(sup_hbm.at[srcs[b]], rows[b], gsems[b]).wait()
      scale(rows[b], ews[b])
      for gi in range(T // L):
        sl = pl.ds(gi * L, L)
        sdsts[b][sl] = dsts[b][sl]
      pltpu.async_copy(rows[b], acc_sh.at[sdsts[b]], ssems[b], add=True)

      @pl.when(k + 2 < NCHUNK)
      def _pi():
        idx_load(k + 2, b, False)

    @pl.loop(0, NCHUNK, step=2)
    def _chunk(k):
      stage(k, 0)

      @pl.when(k + 1 < NCHUNK)
      def _b1():
        stage(k + 1, 1)

    pltpu.make_async_copy(rows1, acc_sh.at[sdst1], ssem1).wait()
    pltpu.make_async_copy(rows0, acc_sh.at[sdst0], ssem0).wait()
    plsc.subcore_barrier()
    for b in range(NB):
      @pl.when(b % NS == sid)
      def _o():
        pltpu.sync_copy(acc_sh.at[pl.ds(b * BR, BR)],
                        out_hbm.at[pl.ds(cid * N + b * BR, BR)])

  return agg(sup, src, dst, ew).reshape(NC, N, d)


def _tc_matmul2(x, W, Ws, b):
  m, k = x.shape
  ds_ = W.shape[1]
  dl = Ws.shape[1]
  bm = 1000

  def body(x_ref, w_ref, ws_ref, b_ref, sup_ref, slf_ref):
    xb = x_ref[...]
    sup_ref[...] = jnp.dot(xb, w_ref[...], preferred_element_type=jnp.float32)
    slf_ref[...] = (jnp.dot(xb, ws_ref[...], preferred_element_type=jnp.float32)
                    + b_ref[...])

  return pl.pallas_call(
      body,
      grid=(m // bm,),
      in_specs=[
          pl.BlockSpec((bm, k), lambda i: (i, 0)),
          pl.BlockSpec((k, ds_), lambda i: (0, 0)),
          pl.BlockSpec((k, dl), lambda i: (0, 0)),
          pl.BlockSpec((1, dl), lambda i: (0, 0)),
      ],
      out_specs=[
          pl.BlockSpec((bm, ds_), lambda i: (i, 0)),
          pl.BlockSpec((bm, dl), lambda i: (i, 0)),
      ],
      out_shape=[jax.ShapeDtypeStruct((m, ds_), jnp.float32),
                 jax.ShapeDtypeStruct((m, dl), jnp.float32)],
  )(x, W, Ws, b.reshape(1, dl))


def _tc_combine_matmul(parts, slf, g, be, W, Ws, b, resid=None):
  n, d = slf.shape
  ds_ = W.shape[1]
  dl = Ws.shape[1]

  def body(*refs):
    if resid is None:
      p_ref, slf_ref, g_ref, be_ref, w_ref, ws_ref, b_ref = refs[:7]
      r_ref = None
    else:
      p_ref, slf_ref, g_ref, be_ref, w_ref, ws_ref, b_ref, r_ref = refs[:8]
    x_ref, sup_ref, slfn_ref = refs[-3:]
    o = p_ref[0][:, :d] + p_ref[1][:, :d] + slf_ref[...]
    mean = jnp.mean(o, axis=0, keepdims=True)
    var = jnp.mean(jnp.square(o - mean), axis=0, keepdims=True)
    o = (o - mean) * lax.rsqrt(var + 1e-5) * g_ref[...] + be_ref[...]
    if r_ref is not None:
      o = o + r_ref[...]
    x_ref[...] = o
    sup_ref[...] = jnp.dot(o, w_ref[...], preferred_element_type=jnp.float32)
    slfn_ref[...] = (jnp.dot(o, ws_ref[...],
                             preferred_element_type=jnp.float32) + b_ref[...])

  args = [parts, slf, g.reshape(1, d), be.reshape(1, d), W, Ws,
          b.reshape(1, dl)]
  if resid is not None:
    args.append(resid)
  return pl.pallas_call(
      body,
      out_shape=[jax.ShapeDtypeStruct((n, d), jnp.float32),
                 jax.ShapeDtypeStruct((n, ds_), jnp.float32),
                 jax.ShapeDtypeStruct((n, dl), jnp.float32)],
  )(*args)


def _tc_combine(parts, slf, g, be, resid=None, softmax=False):
  n, d = slf.shape

  def body(*refs):
    if resid is None:
      p_ref, slf_ref, g_ref, be_ref, out_ref = refs
      r_ref = None
    else:
      p_ref, slf_ref, g_ref, be_ref, r_ref, out_ref = refs
    o = p_ref[0][:, :d] + p_ref[1][:, :d] + slf_ref[...]
    mean = jnp.mean(o, axis=0, keepdims=True)
    var = jnp.mean(jnp.square(o - mean), axis=0, keepdims=True)
    o = (o - mean) * lax.rsqrt(var + 1e-5) * g_ref[...] + be_ref[...]
    if r_ref is not None:
      o = o + r_ref[...]
    if softmax:
      mx = jnp.max(o, axis=1, keepdims=True)
      o = o - mx
      o = o - jnp.log(jnp.sum(jnp.exp(o), axis=1, keepdims=True))
    out_ref[...] = o

  args = [parts, slf, g.reshape(1, d), be.reshape(1, d)]
  if resid is not None:
    args.append(resid)
  return pl.pallas_call(
      body,
      out_shape=jax.ShapeDtypeStruct((n, d), jnp.float32),
  )(*args)


def kernel(fea, edge_index, edge_weight,
           W0, Ws0, b0, g0, be0,
           W1, Ws1, b1, g1, be1,
           W2, Ws2, b2, g2, be2,
           W3, Ws3, b3, g3, be3):
  src = edge_index[0]
  dst = edge_index[1]
  ew = edge_weight

  W3p = jnp.pad(W3, ((0, 0), (0, 128 - W3.shape[1])))

  sup, slf = _tc_matmul2(fea, W0, Ws0, b0)
  parts = _sc_aggregate(sup, src, dst, ew, d=128)
  x0, sup, slf = _tc_combine_matmul(parts, slf, g0, be0, W1, Ws1, b1)
  parts = _sc_aggregate(sup, src, dst, ew, d=128)
  _, sup, slf = _tc_combine_matmul(parts, slf, g1, be1, W2, Ws2, b2)
  parts = _sc_aggregate(sup, src, dst, ew, d=128)
  _, sup, slf = _tc_combine_matmul(parts, slf, g2, be2, W3p, Ws3, b3,
                                   resid=x0)
  parts = _sc_aggregate(sup, src, dst, ew, d=128)
  return _tc_combine(parts, slf, g3, be3, softmax=True)
